# Initial kernel scaffold; baseline (speedup 1.0000x reference)
import jax
import jax.numpy as jnp
from jax.experimental import pallas as pl


def kernel(input_ids, tok_table, pos_table):
    raise NotImplementedError("write your pallas kernel here")


# speedup vs baseline: 1.5846x; 1.5846x over previous
import functools

import jax
import jax.numpy as jnp
from jax import lax
from jax.experimental import pallas as pl
from jax.experimental.pallas import tpu as pltpu
from jax.experimental.pallas import tpu_sc as plsc

VOCAB = 50272
D = 768
MAX_POS = 2048
OFFSET = 2
B, S = 4, 2048

NC, NS = 2, 16
NW = NC * NS
LANES = 16
SPW = S // NW
CCHUNKS = D // LANES


def _emb_body(ids_hbm, tok_hbm, pos_hbm, out_hbm, idx_v, pos_v, tok_v, sem):
    wid = lax.axis_index("s") * NC + lax.axis_index("c")
    s0 = wid * SPW

    for c in range(SPW // LANES):
        idx_v[pl.ds(c * LANES, LANES)] = (
            lax.iota(jnp.int32, LANES) + (OFFSET + s0 + c * LANES)
        )
    pltpu.async_copy(pos_hbm.at[idx_v], pos_v, sem).wait()

    for b in range(B):
        pltpu.sync_copy(ids_hbm.at[pl.ds(b * S + s0, SPW)], idx_v)
        pltpu.async_copy(tok_hbm.at[idx_v], tok_v, sem).wait()

        def row_body(r, _):
            def col_body(c, _):
                sl = pl.ds(c * LANES, LANES)
                tok_v[r, sl] = tok_v[r, sl] + pos_v[r, sl]
                return 0

            return lax.fori_loop(0, CCHUNKS, col_body, 0, unroll=8)

        lax.fori_loop(0, SPW, row_body, 0)
        pltpu.sync_copy(tok_v, out_hbm.at[b, pl.ds(s0, SPW)])


@jax.jit
def _emb(input_ids, tok_table, pos_table):
    mesh = plsc.VectorSubcoreMesh(core_axis_name="c", subcore_axis_name="s")
    return pl.kernel(
        _emb_body,
        out_type=jax.ShapeDtypeStruct((B, S, D), jnp.float32),
        mesh=mesh,
        scratch_types=[
            pltpu.VMEM((SPW,), jnp.int32),
            pltpu.VMEM((SPW, D), jnp.float32),
            pltpu.VMEM((SPW, D), jnp.float32),
            pltpu.SemaphoreType.DMA,
        ],
    )(input_ids, tok_table, pos_table)


def kernel(input_ids, tok_table, pos_table):
    ids_flat = input_ids.astype(jnp.int32).reshape(B * S)
    return _emb(ids_flat, tok_table, pos_table)


# speedup vs baseline: 3.1101x; 1.9627x over previous
import jax
import jax.numpy as jnp
from jax import lax
from jax.experimental import pallas as pl
from jax.experimental.pallas import tpu as pltpu
from jax.experimental.pallas import tpu_sc as plsc

VOCAB = 50272
D = 768
MAX_POS = 2048
OFFSET = 2
B, S = 4, 2048

NC, NS = 2, 16
NW = NC * NS
LANES = 16
SPW = S // NW
CH = 32
NCHUNK = B * SPW // CH
CCHUNKS = D // LANES


def _emb_body(ids_hbm, tok_hbm, pos_hbm, out_hbm,
              idx_all, pos_idx, pos_v, buf0, buf1,
              sem_p, sem_i, sem_g0, sem_g1, sem_o0, sem_o1):
    wid = lax.axis_index("s") * NC + lax.axis_index("c")
    s0 = wid * SPW
    bufs = (buf0, buf1)
    sem_g = (sem_g0, sem_g1)
    sem_o = (sem_o0, sem_o1)

    for c in range(SPW // LANES):
        pos_idx[pl.ds(c * LANES, LANES)] = (
            lax.iota(jnp.int32, LANES) + (OFFSET + s0 + c * LANES)
        )
    pos_cp = pltpu.async_copy(pos_hbm.at[pos_idx], pos_v, sem_p)

    id_cps = [
        pltpu.async_copy(ids_hbm.at[pl.ds(b * S + s0, SPW)],
                         idx_all.at[pl.ds(b * SPW, SPW)], sem_i)
        for b in range(B)
    ]
    for cp in id_cps:
        cp.wait()

    gather = {}

    def start_gather(k):
        gather[k] = pltpu.async_copy(
            tok_hbm.at[idx_all.at[pl.ds(k * CH, CH)]], bufs[k & 1],
            sem_g[k & 1])

    start_gather(0)
    pos_cp.wait()

    out_cp = {}
    for k in range(NCHUNK):
        h = k & 1
        if k + 1 < NCHUNK:
            if k >= 1:
                out_cp[k - 1].wait()
            start_gather(k + 1)
        gather[k].wait()

        buf = bufs[h]

        @plsc.parallel_loop(0, CH)
        def _add_row(r):
            for c in range(CCHUNKS):
                sl = pl.ds(c * LANES, LANES)
                plsc.addupdate(buf.at[r, sl], pos_v[h * CH + r, sl])

        out_cp[k] = pltpu.async_copy(
            buf, out_hbm.at[k >> 1, pl.ds(s0 + h * CH, CH)], sem_o[h])

    out_cp[NCHUNK - 2].wait()
    out_cp[NCHUNK - 1].wait()


@jax.jit
def _emb(ids_flat, tok_table, pos_table):
    mesh = plsc.VectorSubcoreMesh(core_axis_name="c", subcore_axis_name="s")
    return pl.kernel(
        _emb_body,
        out_type=jax.ShapeDtypeStruct((B, S, D), jnp.float32),
        mesh=mesh,
        scratch_types=[
            pltpu.VMEM((B * SPW,), jnp.int32),
            pltpu.VMEM((SPW,), jnp.int32),
            pltpu.VMEM((SPW, D), jnp.float32),
            pltpu.VMEM((CH, D), jnp.float32),
            pltpu.VMEM((CH, D), jnp.float32),
            pltpu.SemaphoreType.DMA,
            pltpu.SemaphoreType.DMA,
            pltpu.SemaphoreType.DMA,
            pltpu.SemaphoreType.DMA,
            pltpu.SemaphoreType.DMA,
            pltpu.SemaphoreType.DMA,
        ],
    )(ids_flat, tok_table, pos_table)


def kernel(input_ids, tok_table, pos_table):
    ids_flat = input_ids.astype(jnp.int32).reshape(B * S)
    return _emb(ids_flat, tok_table, pos_table)


# speedup vs baseline: 3.5570x; 1.1437x over previous
import jax
import jax.numpy as jnp
from jax import lax
from jax.experimental import pallas as pl
from jax.experimental.pallas import tpu as pltpu
from jax.experimental.pallas import tpu_sc as plsc

VOCAB = 50272
D = 768
MAX_POS = 2048
OFFSET = 2
B, S = 4, 2048

NC, NS = 2, 16
NW = NC * NS
LANES = 16
SPW = S // NW
CH = 16
NBUF = 4
CCHUNKS = D // LANES


def _emb_body(ids_hbm, tok_hbm, pos_hbm, out_hbm,
              idx_full, pos_idx, pos_v, buf0, buf1, buf2, buf3,
              sem_p, sem_i, sem_g0, sem_g1, sem_g2, sem_g3,
              sem_o0, sem_o1, sem_o2, sem_o3):
    wid = lax.axis_index("s") * NC + lax.axis_index("c")
    s0 = wid * SPW
    bufs = (buf0, buf1, buf2, buf3)
    sem_g = (sem_g0, sem_g1, sem_g2, sem_g3)
    sem_o = (sem_o0, sem_o1, sem_o2, sem_o3)

    id_cp = pltpu.async_copy(ids_hbm, idx_full, sem_i)

    for c in range(SPW // LANES):
        pos_idx[pl.ds(c * LANES, LANES)] = (
            lax.iota(jnp.int32, LANES) + (OFFSET + s0 + c * LANES)
        )
    pos_cp = pltpu.async_copy(pos_hbm.at[pos_idx], pos_v, sem_p)

    def g_start(b, q):
        pltpu.async_copy(
            tok_hbm.at[idx_full.at[b, pl.ds(s0 + q * CH, CH)]],
            bufs[q % NBUF], sem_g[q % NBUF])

    def g_wait(m):
        pltpu.make_async_copy(
            tok_hbm.at[idx_full.at[0, pl.ds(s0, CH)]], bufs[m],
            sem_g[m]).wait()

    def o_start(b, q):
        pltpu.async_copy(bufs[q], out_hbm.at[b, pl.ds(s0 + q * CH, CH)],
                         sem_o[q])

    def o_wait(m):
        pltpu.make_async_copy(bufs[m], out_hbm.at[0, pl.ds(s0, CH)],
                              sem_o[m]).wait()

    id_cp.wait()
    g_start(0, 0)
    g_start(0, 1)
    pos_cp.wait()

    @pl.loop(0, B)
    def _row(b):
        for j in range(NBUF):
            g_wait(j)
            if j < 2:
                @pl.when(b > 0)
                def _():
                    o_wait(j + 2)

                g_start(b, j + 2)
            else:
                @pl.when(b < B - 1)
                def _():
                    o_wait(j - 2)
                    g_start(b + 1, j - 2)

            buf = bufs[j]

            @plsc.parallel_loop(0, CH)
            def _add_row(r):
                for c in range(CCHUNKS):
                    sl = pl.ds(c * LANES, LANES)
                    plsc.addupdate(buf.at[r, sl], pos_v[j * CH + r, sl])

            o_start(b, j)

    for m in range(NBUF):
        o_wait(m)


@jax.jit
def _emb(input_ids, tok_table, pos_table):
    mesh = plsc.VectorSubcoreMesh(core_axis_name="c", subcore_axis_name="s")
    return pl.kernel(
        _emb_body,
        out_type=jax.ShapeDtypeStruct((B, S, D), jnp.float32),
        mesh=mesh,
        scratch_types=[
            pltpu.VMEM((B, S), jnp.int32),
            pltpu.VMEM((SPW,), jnp.int32),
            pltpu.VMEM((SPW, D), jnp.float32),
            pltpu.VMEM((CH, D), jnp.float32),
            pltpu.VMEM((CH, D), jnp.float32),
            pltpu.VMEM((CH, D), jnp.float32),
            pltpu.VMEM((CH, D), jnp.float32),
        ] + [pltpu.SemaphoreType.DMA] * 10,
    )(input_ids, tok_table, pos_table)


def kernel(input_ids, tok_table, pos_table):
    return _emb(input_ids.astype(jnp.int32), tok_table, pos_table)
